# baseline (speedup 1.0000x reference)
import jax
import jax.numpy as jnp
from jax import lax
from jax.experimental import pallas as pl
from jax.experimental.pallas import tpu as pltpu
from jax.experimental.pallas import tpu_sc as plsc

N = 8192
ROWS = 64
RB = 256
GRID = N // RB

THRESHOLD = 0.3
GAMMA = 1.0


def _smape_key(pre, tgt):
    loss = 2.0 * jnp.abs(tgt - pre) / (jnp.abs(tgt) + jnp.abs(pre))
    valid = jnp.logical_not(jnp.isnan(tgt)) & (tgt != 0)
    loss = jnp.where(valid, loss, jnp.float32(3.0))
    key = lax.bitcast_convert_type(loss, jnp.int32)
    return loss, key


def _rank_body(pre_r, tgt_r, pre_c, tgt_c, rank_ref, loss_ref):
    pid = pl.program_id(0)
    _, key_r = _smape_key(pre_r[:], tgt_r[:])
    loss_c, key_c = _smape_key(pre_c[:], tgt_c[:])
    iidx = pid * RB + lax.broadcasted_iota(jnp.int32, (RB, 1), 0)
    acc = jnp.zeros((RB, 128), jnp.int32)
    for kc in range(ROWS):
        kk = key_r[kc, :][None, :]
        kidx = kc * 128 + lax.broadcasted_iota(jnp.int32, (1, 128), 1)
        lt = kk < key_c
        tie = (kk == key_c) & (kidx < iidx)
        acc = acc + (lt | tie).astype(jnp.int32)
    rank_ref[:] = jnp.sum(acc, axis=1, keepdims=True)
    loss_ref[:] = loss_c


def _spl_body(a_r, lvl_r, a_c, lvl_c, loss_c, out_ref):
    comp_r = lvl_r[:] * N + a_r[:]
    comp_c = lvl_c[:] * N + a_c[:]
    acc = jnp.zeros((RB, 128), jnp.int32)
    for kc in range(ROWS):
        ck = comp_r[kc, :][None, :]
        acc = acc + (ck < comp_c).astype(jnp.int32)
    rank2 = jnp.sum(acc, axis=1, keepdims=True)
    lvlr = lvl_r[:]
    starts = jnp.zeros((RB, 1), jnp.int32)
    for level in range(4):
        cnt = jnp.sum((lvlr == level).astype(jnp.int32))
        starts = starts + jnp.where(lvl_c[:] > level, cnt, 0)
    spl = (rank2 - starts).astype(jnp.float32)
    thr = THRESHOLD + GAMMA / (jnp.sqrt(spl + 1.0) + jnp.sqrt(spl))
    out_ref[:] = jnp.where(loss_c[:] < thr, 0.0, 1.0)


_row_spec = pl.BlockSpec((ROWS, 128), lambda i: (0, 0))
_col_spec = pl.BlockSpec((RB, 1), lambda i: (i, 0))

_rank_call = pl.pallas_call(
    _rank_body,
    grid=(GRID,),
    in_specs=[_row_spec, _row_spec, _col_spec, _col_spec],
    out_specs=[_col_spec, _col_spec],
    out_shape=[
        jax.ShapeDtypeStruct((N, 1), jnp.int32),
        jax.ShapeDtypeStruct((N, 1), jnp.float32),
    ],
)

_spl_call = pl.pallas_call(
    _spl_body,
    grid=(GRID,),
    in_specs=[_row_spec, _row_spec, _col_spec, _col_spec, _col_spec],
    out_specs=_col_spec,
    out_shape=jax.ShapeDtypeStruct((N, 1), jnp.float32),
)

_NC = 2
_NS = 16
_NW = _NC * _NS
_ROWS_PER_W = ROWS // _NW


def _invert_body(rank_hbm, out_hbm, idx_v, val_v, sem):
    c = lax.axis_index("c")
    s = lax.axis_index("s")
    wid = s * _NC + c
    row0 = wid * _ROWS_PER_W
    pltpu.sync_copy(rank_hbm.at[pl.ds(row0, _ROWS_PER_W)], idx_v)
    for j in range(_ROWS_PER_W):
        gbase = (row0 + j) * 128
        for t in range(8):
            val_v[j, pl.ds(t * 16, 16)] = gbase + t * 16 + lax.iota(jnp.int32, 16)
    for j in range(_ROWS_PER_W):
        pltpu.async_copy(val_v.at[j], out_hbm.at[idx_v.at[j]], sem).wait()


def _invert_call(rank_rows):
    call = pl.kernel(
        _invert_body,
        out_type=jax.ShapeDtypeStruct((N,), jnp.int32),
        mesh=plsc.VectorSubcoreMesh(
            core_axis_name="c", subcore_axis_name="s",
            num_cores=_NC, num_subcores=_NS,
        ),
        scratch_types=[
            pltpu.VMEM((_ROWS_PER_W, 128), jnp.int32),
            pltpu.VMEM((_ROWS_PER_W, 128), jnp.int32),
            pltpu.SemaphoreType.DMA,
        ],
    )
    return call(rank_rows)


def kernel(sta_pre, sta_tgt, sta_level):
    pre = sta_pre.reshape(N)
    tgt = sta_tgt.reshape(N)
    lvl = sta_level.reshape(N).astype(jnp.int32)

    rank, loss = _rank_call(
        pre.reshape(ROWS, 128), tgt.reshape(ROWS, 128),
        pre.reshape(N, 1), tgt.reshape(N, 1),
    )
    a = _invert_call(rank.reshape(ROWS, 128))
    out = _spl_call(
        a.reshape(ROWS, 128), lvl.reshape(ROWS, 128),
        a.reshape(N, 1), lvl.reshape(N, 1), loss,
    )
    return out.reshape(N)


# speedup vs baseline: 1.4480x; 1.4480x over previous
import jax
import jax.numpy as jnp
from jax import lax
from jax.experimental import pallas as pl
from jax.experimental.pallas import tpu as pltpu
from jax.experimental.pallas import tpu_sc as plsc

N = 8192
ROWS = 64
BR = 2
GRID = ROWS // BR

THRESHOLD = 0.3
GAMMA = 1.0

NLVL = 5


def _rank_body(key_r, key_b, rank_ref, key_sref):
    pid = pl.program_id(0)

    @pl.when(pid == 0)
    def _():
        key_sref[:] = key_r[:]

    kb = key_b[0]
    ki = kb[:, :, None]
    acc0 = jnp.zeros((BR, 128, 128), jnp.int32)

    def body_le(kc, acc):
        kk = key_sref[pl.ds(kc, 1), :][None]
        return acc + (kk <= ki).astype(jnp.int32)

    def body_lt(kc, acc):
        kk = key_sref[pl.ds(kc, 1), :][None]
        return acc + (kk < ki).astype(jnp.int32)

    acc = lax.fori_loop(0, BR * pid, body_le, acc0)
    acc = lax.fori_loop(BR * pid + BR, ROWS, body_lt, acc)

    ilane = lax.broadcasted_iota(jnp.int32, (BR, 128, 128), 1)
    klane = lax.broadcasted_iota(jnp.int32, (BR, 128, 128), 2)
    irow = lax.broadcasted_iota(jnp.int32, (BR, 128, 128), 0)
    for r in range(BR):
        kk = kb[r][None, None, :]
        lt = kk < ki
        eq = kk == ki
        idxlt = (r < irow) | ((r == irow) & (klane < ilane))
        acc = acc + (lt | (eq & idxlt)).astype(jnp.int32)

    rank_ref[0] = jnp.sum(acc, axis=2)


_row_spec = pl.BlockSpec((ROWS, 128), lambda i: (0, 0))
_blk_spec = pl.BlockSpec((1, BR, 128), lambda i: (i, 0, 0))

_rank_call = pl.pallas_call(
    _rank_body,
    grid=(GRID,),
    in_specs=[_row_spec, _blk_spec],
    out_specs=_blk_spec,
    out_shape=jax.ShapeDtypeStruct((GRID, BR, 128), jnp.int32),
    scratch_shapes=[pltpu.VMEM((ROWS, 128), jnp.int32)],
)

_NC = 2
_NS = 16
_EPW = N // _NS
_RPW = _EPW // 128


def _sc_body(rank_hbm, lvl_hbm, out_hbm, rk_v, m_v, s_v, smem_off, sem):
    core = lax.axis_index("c")
    sub = lax.axis_index("s")

    @pl.when(core == 0)
    def _():
        for level in range(NLVL):
            smem_off[level] = jnp.int32(0)
        plsc.subcore_barrier()

        base = sub * _EPW
        for r in range(_RPW):
            pltpu.sync_copy(rank_hbm.at[pl.ds(base + r * 128, 128)], rk_v.at[r])
        for r in range(_RPW):
            pltpu.async_copy(lvl_hbm.at[rk_v.at[r]], m_v.at[r], sem).wait()

        lane = lax.iota(jnp.int32, 16)
        carr16 = jnp.zeros((16,), jnp.int32)
        for r in range(_RPW):
            for c in range(8):
                m16 = m_v[r, pl.ds(c * 16, 16)]
                s16 = jnp.zeros((16,), jnp.int32)
                for level in range(NLVL):
                    s16 = s16 + jnp.where(m16 == level, carr16[level], 0)
                hist = jnp.zeros((16,), jnp.int32)
                for lp in range(16):
                    mv = m16[lp]
                    s16 = s16 + jnp.where((lane > lp) & (m16 == mv), 1, 0)
                    hist = hist + jnp.where(lane == mv, 1, 0)
                carr16 = carr16 + hist
                s_v[r, pl.ds(c * 16, 16)] = s16

        totals = [carr16[level] for level in range(NLVL)]

        def publish(v, carry):
            for level in range(NLVL):
                plsc.fetch_and_add(smem_off.at[level], totals[level],
                                   subcore_id=v)
            return carry

        lax.fori_loop(sub + 1, _NS, publish, jnp.int32(0))
        plsc.subcore_barrier()
        offs = [smem_off[level] for level in range(NLVL)]

        for r in range(_RPW):
            for c in range(8):
                m16 = m_v[r, pl.ds(c * 16, 16)]
                s16 = s_v[r, pl.ds(c * 16, 16)]
                for level in range(NLVL):
                    s16 = s16 + jnp.where(m16 == level, offs[level], 0)
                s_v[r, pl.ds(c * 16, 16)] = s16
        for r in range(_RPW):
            pltpu.async_copy(s_v.at[r], out_hbm.at[rk_v.at[r]], sem).wait()


def _sc_call(rank, lvl):
    call = pl.kernel(
        _sc_body,
        out_type=jax.ShapeDtypeStruct((N,), jnp.int32),
        mesh=plsc.VectorSubcoreMesh(
            core_axis_name="c", subcore_axis_name="s",
            num_cores=_NC, num_subcores=_NS,
        ),
        scratch_types=[
            pltpu.VMEM((_RPW, 128), jnp.int32),
            pltpu.VMEM((_RPW, 128), jnp.int32),
            pltpu.VMEM((_RPW, 128), jnp.int32),
            pltpu.SMEM((8,), jnp.int32),
            pltpu.SemaphoreType.DMA,
        ],
    )
    return call(rank, lvl)


def kernel(sta_pre, sta_tgt, sta_level):
    tgt = sta_tgt[:, :, :1]
    loss_smape = 2.0 * jnp.abs(tgt - sta_pre) / (jnp.abs(tgt) + jnp.abs(sta_pre))
    loss_zero = jnp.ones_like(tgt) * 3.0
    valid = (~jnp.isnan(tgt)) & (tgt != 0)
    loss_smape = jnp.where(valid, loss_smape, loss_zero)
    loss = loss_smape.reshape(-1)

    key = lax.bitcast_convert_type(loss, jnp.int32)
    lvl = sta_level.reshape(N).astype(jnp.int32)

    rank = _rank_call(key.reshape(ROWS, 128), key.reshape(GRID, BR, 128))
    spl = _sc_call(rank.reshape(N), lvl).astype(jnp.float32)

    v = (loss < THRESHOLD + GAMMA / (jnp.sqrt(spl + 1.0) + jnp.sqrt(spl))
         ).astype(jnp.float32)
    return 1.0 - v
